# Initial kernel scaffold; baseline (speedup 1.0000x reference)
import jax
import jax.numpy as jnp
from jax.experimental import pallas as pl


def kernel(ids, feats, edge_dict, G, ite, W, b):
    raise NotImplementedError("write your pallas kernel here")


# speedup vs baseline: 1.0879x; 1.0879x over previous
import functools

import jax
import jax.numpy as jnp
from jax import lax
from jax.experimental import pallas as pl
from jax.experimental.pallas import tpu as pltpu
from jax.experimental.pallas import tpu_sc as plsc

N = 10000
K = 32
DIN = 128
DOUT = 128

NC = 2
NS = 16
NW = NC * NS
NPW = 320
NPAD = NW * NPW
CH = 4
NCHUNK = NPW // CH
LANES = 16


def _fc_body(x_ref, w_ref, b_ref, h_ref):
    acc = lax.dot_general(x_ref[...], w_ref[...],
                          (((1,), (1,)), ((), ())),
                          preferred_element_type=jnp.float32)
    h_ref[...] = jnp.maximum(acc + b_ref[...], 0.0)


def _fc(feats, W, b2):
    blk = 1000
    return pl.pallas_call(
        _fc_body,
        grid=(N // blk,),
        in_specs=[
            pl.BlockSpec((blk, DIN), lambda i: (i, 0)),
            pl.BlockSpec((DOUT, DIN), lambda i: (0, 0)),
            pl.BlockSpec((1, DOUT), lambda i: (0, 0)),
        ],
        out_specs=pl.BlockSpec((blk, DOUT), lambda i: (i, 0)),
        out_shape=jax.ShapeDtypeStruct((N, DOUT), jnp.float32),
    )(feats, W, b2)


def _pool_body(h_hbm, edge_hbm, out_hbm, idx_v, rows_v, out_v, sem):
    cid = lax.axis_index("c")
    sid = lax.axis_index("s")
    wid = sid * NC + cid
    node_base = wid * NPW

    def chunk(c, _):
        ebase = pl.multiple_of((node_base + c * CH) * K, CH * K)
        pltpu.sync_copy(edge_hbm.at[pl.ds(ebase, CH * K)], idx_v)
        pltpu.async_copy(h_hbm.at[idx_v], rows_v, sem).wait()
        inv = jnp.full((LANES,), 1.0 / K, dtype=jnp.float32)
        for n in range(CH):
            row = c * CH + n
            for d in range(DOUT // LANES):
                sl = pl.ds(d * LANES, LANES)
                acc = rows_v[n * K, sl]
                for j in range(1, K):
                    acc = acc + rows_v[n * K + j, sl]
                out_v[row, sl] = acc * inv
        return _

    lax.fori_loop(0, NCHUNK, chunk, None)
    pltpu.sync_copy(out_v, out_hbm.at[pl.ds(node_base, NPW)])


def _pool(h, edge_flat):
    mesh = plsc.VectorSubcoreMesh(core_axis_name="c", subcore_axis_name="s")
    f = pl.kernel(
        _pool_body,
        out_type=jax.ShapeDtypeStruct((NPAD, DOUT), jnp.float32),
        mesh=mesh,
        scratch_types=[
            pltpu.VMEM((CH * K,), jnp.int32),
            pltpu.VMEM((CH * K, DOUT), jnp.float32),
            pltpu.VMEM((NPW, DOUT), jnp.float32),
            pltpu.SemaphoreType.DMA,
        ],
    )
    return f(h, edge_flat)


def kernel(ids, feats, edge_dict, G, ite, W, b):
    h = _fc(feats, W, b.reshape(1, DOUT))
    edge_flat = jnp.pad(edge_dict.reshape(-1), (0, (NPAD - N) * K))
    pooled = _pool(h, edge_flat)
    return pooled[:N]


# speedup vs baseline: 1.5576x; 1.4318x over previous
import functools

import jax
import jax.numpy as jnp
from jax import lax
from jax.experimental import pallas as pl
from jax.experimental.pallas import tpu as pltpu
from jax.experimental.pallas import tpu_sc as plsc

N = 10000
K = 32
DIN = 128
DOUT = 128

NC = 2
NS = 16
NW = NC * NS
NPW = 320
NPAD = NW * NPW
CH = 4
NCHUNK = NPW // CH
LANES = 16
NBUF = 2


def _fc_body(x_ref, w_ref, b_ref, h_ref):
    acc = lax.dot_general(x_ref[...], w_ref[...],
                          (((1,), (1,)), ((), ())),
                          preferred_element_type=jnp.float32)
    h_ref[...] = jnp.maximum(acc + b_ref[...], 0.0)


def _fc(feats, W, b2):
    blk = 1000
    return pl.pallas_call(
        _fc_body,
        grid=(N // blk,),
        in_specs=[
            pl.BlockSpec((blk, DIN), lambda i: (i, 0)),
            pl.BlockSpec((DOUT, DIN), lambda i: (0, 0)),
            pl.BlockSpec((1, DOUT), lambda i: (0, 0)),
        ],
        out_specs=pl.BlockSpec((blk, DOUT), lambda i: (i, 0)),
        out_shape=jax.ShapeDtypeStruct((N, DOUT), jnp.float32),
    )(feats, W, b2)


def _pool_body(h_hbm, edge_hbm, out_hbm, idx_all, rows0, rows1, out_v,
               sem0, sem1):
    cid = lax.axis_index("c")
    sid = lax.axis_index("s")
    wid = sid * NC + cid
    node_base = wid * NPW

    pltpu.sync_copy(edge_hbm.at[pl.ds(wid * NCHUNK, NCHUNK)], idx_all)

    rows = (rows0, rows1)
    sems = (sem0, sem1)
    for b in range(NBUF):
        pltpu.async_copy(h_hbm.at[idx_all.at[b]], rows[b], sems[b])

    inv = jnp.full((LANES,), 1.0 / K, dtype=jnp.float32)

    def step(g, carry):
        for b in range(NBUF):
            c = g * NBUF + b
            r = rows[b]
            pltpu.make_async_copy(h_hbm.at[idx_all.at[c]], r, sems[b]).wait()
            for n in range(CH):
                row = c * CH + n
                for d in range(DOUT // LANES):
                    sl = pl.ds(d * LANES, LANES)
                    acc = r[n * K, sl]
                    for j in range(1, K):
                        acc = acc + r[n * K + j, sl]
                    out_v[row, sl] = acc * inv

            @pl.when(c + NBUF < NCHUNK)
            def _():
                pltpu.async_copy(h_hbm.at[idx_all.at[c + NBUF]], r, sems[b])
        return carry

    lax.fori_loop(0, NCHUNK // NBUF, step, None)
    pltpu.sync_copy(out_v, out_hbm.at[pl.ds(node_base, NPW)])


def _pool(h, edge2):
    mesh = plsc.VectorSubcoreMesh(core_axis_name="c", subcore_axis_name="s")
    f = pl.kernel(
        _pool_body,
        out_type=jax.ShapeDtypeStruct((NPAD, DOUT), jnp.float32),
        mesh=mesh,
        scratch_types=[
            pltpu.VMEM((NCHUNK, CH * K), jnp.int32),
            pltpu.VMEM((CH * K, DOUT), jnp.float32),
            pltpu.VMEM((CH * K, DOUT), jnp.float32),
            pltpu.VMEM((NPW, DOUT), jnp.float32),
            pltpu.SemaphoreType.DMA,
            pltpu.SemaphoreType.DMA,
        ],
    )
    return f(h, edge2)


def kernel(ids, feats, edge_dict, G, ite, W, b):
    h = _fc(feats, W, b.reshape(1, DOUT))
    edge_flat = jnp.pad(edge_dict.reshape(-1), (0, (NPAD - N) * K))
    edge2 = edge_flat.reshape(NW * NCHUNK, CH * K)
    pooled = _pool(h, edge2)
    return pooled[:N]


# speedup vs baseline: 2.2965x; 1.4744x over previous
import functools

import jax
import jax.numpy as jnp
from jax import lax
from jax.experimental import pallas as pl
from jax.experimental.pallas import tpu as pltpu
from jax.experimental.pallas import tpu_sc as plsc

N = 10000
K = 32
DIN = 128
DOUT = 128
DH = DOUT // 2

NC = 2
NS = 16
NW = NC * NS
NPW = 320
NPAD = NW * NPW
CH = 4
NCHUNK = NPW // CH
LANES = 16
NBUF = 2
STRIPE = 640


def _fc_body(x_ref, w_ref, b_ref, h_ref):
    acc = lax.dot_general(x_ref[...], w_ref[...],
                          (((1,), (1,)), ((), ())),
                          preferred_element_type=jnp.float32)
    h = jnp.maximum(acc + b_ref[...], 0.0)
    lo = lax.bitcast_convert_type(
        h[:, :DH].astype(jnp.bfloat16), jnp.uint16).astype(jnp.uint32)
    hi = lax.bitcast_convert_type(
        h[:, DH:].astype(jnp.bfloat16), jnp.uint16).astype(jnp.uint32)
    h_ref[...] = lax.bitcast_convert_type((hi << 16) | lo, jnp.float32)


def _fc(feats, W, b2):
    blk = 1000
    return pl.pallas_call(
        _fc_body,
        grid=(N // blk,),
        in_specs=[
            pl.BlockSpec((blk, DIN), lambda i: (i, 0)),
            pl.BlockSpec((DOUT, DIN), lambda i: (0, 0)),
            pl.BlockSpec((1, DOUT), lambda i: (0, 0)),
        ],
        out_specs=pl.BlockSpec((blk, DH), lambda i: (i, 0)),
        out_shape=jax.ShapeDtypeStruct((N, DH), jnp.float32),
    )(feats, W, b2)


def _pool_body(h_hbm, edge_hbm, out_hbm, idx_all, rows0, rows1, out_v,
               sem0, sem1):
    cid = lax.axis_index("c")
    sid = lax.axis_index("s")
    wid = sid * NC + cid
    node_base = wid * NPW

    pltpu.sync_copy(edge_hbm.at[pl.ds(wid * NCHUNK, NCHUNK)], idx_all)

    rows = (rows0, rows1)
    sems = (sem0, sem1)
    for b in range(NBUF):
        pltpu.async_copy(h_hbm.at[idx_all.at[b]], rows[b], sems[b])

    inv = jnp.full((LANES,), 1.0 / K, dtype=jnp.float32)

    def step(g, carry):
        for b in range(NBUF):
            c = g * NBUF + b
            r = rows[b]
            pltpu.make_async_copy(h_hbm.at[idx_all.at[c]], r, sems[b]).wait()
            for n in range(CH):
                row = c * CH + n
                for w in range(DH // LANES):
                    sl = pl.ds(w * LANES, LANES)
                    acc_lo = jnp.zeros((LANES,), jnp.float32)
                    acc_hi = jnp.zeros((LANES,), jnp.float32)
                    for j in range(K):
                        packed = plsc.bitcast(r[n * K + j, sl], jnp.bfloat16)
                        lo, hi = plsc.unpack(
                            packed, format=plsc.PackFormat.INTERLEAVED,
                            preferred_element_type=jnp.float32)
                        acc_lo = acc_lo + lo
                        acc_hi = acc_hi + hi
                    out_v[row, sl] = acc_lo * inv
                    out_v[row, pl.ds(DH + w * LANES, LANES)] = acc_hi * inv

            @pl.when(c + NBUF < NCHUNK)
            def _():
                pltpu.async_copy(h_hbm.at[idx_all.at[c + NBUF]], r, sems[b])
        return carry

    lax.fori_loop(0, NCHUNK // NBUF, step, None)
    pltpu.sync_copy(out_v, out_hbm.at[pl.ds(node_base, NPW)])


def _pool(h, edge2):
    mesh = plsc.VectorSubcoreMesh(core_axis_name="c", subcore_axis_name="s")
    f = pl.kernel(
        _pool_body,
        out_type=jax.ShapeDtypeStruct((NPAD, DOUT), jnp.float32),
        mesh=mesh,
        compiler_params=pltpu.CompilerParams(needs_layout_passes=False,
                                             use_tc_tiling_on_sc=False),
        scratch_types=[
            pltpu.VMEM((NCHUNK, CH * K), jnp.int32),
            pltpu.VMEM((CH * K, DH), jnp.float32),
            pltpu.VMEM((CH * K, DH), jnp.float32),
            pltpu.VMEM((NPW, DOUT), jnp.float32),
            pltpu.SemaphoreType.DMA,
            pltpu.SemaphoreType.DMA,
        ],
    )
    return f(h, edge2)


def kernel(ids, feats, edge_dict, G, ite, W, b):
    h = _fc(feats, W, b.reshape(1, DOUT))
    edge_flat = jnp.pad(edge_dict.reshape(-1), (0, (NPAD - N) * K))
    edge2 = edge_flat.reshape(NW * NCHUNK, CH * K)
    pooled = _pool(h, edge2)
    return pooled[:N]


# speedup vs baseline: 2.4732x; 1.0769x over previous
import functools

import jax
import jax.numpy as jnp
from jax import lax
from jax.experimental import pallas as pl
from jax.experimental.pallas import tpu as pltpu
from jax.experimental.pallas import tpu_sc as plsc

N = 10000
K = 32
DIN = 128
DOUT = 128
DH = DOUT // 2

NC = 2
NS = 16
NW = NC * NS
NPAD = 10240
CH = 4
NCHUNKS_TOTAL = NPAD // CH
LANES = 16
NBUF = 2

CHUNKS0 = 106
CHUNKS1 = 54
NODES0 = CHUNKS0 * CH
NODES1 = CHUNKS1 * CH
CORE0_NODES = NS * NODES0
CORE0_CHUNKS = NS * CHUNKS0
EDGE_ROWS = 2624


def _fc_body(x_ref, w_ref, b_ref, h_ref):
    acc = lax.dot_general(x_ref[...], w_ref[...],
                          (((1,), (1,)), ((), ())),
                          preferred_element_type=jnp.float32)
    h = jnp.maximum(acc + b_ref[...], 0.0)
    lo = lax.bitcast_convert_type(
        h[:, :DH].astype(jnp.bfloat16), jnp.uint16).astype(jnp.uint32)
    hi = lax.bitcast_convert_type(
        h[:, DH:].astype(jnp.bfloat16), jnp.uint16).astype(jnp.uint32)
    h_ref[...] = lax.bitcast_convert_type((hi << 16) | lo, jnp.float32)


def _fc(feats, W, b2):
    blk = 1000
    return pl.pallas_call(
        _fc_body,
        grid=(N // blk,),
        in_specs=[
            pl.BlockSpec((blk, DIN), lambda i: (i, 0)),
            pl.BlockSpec((DOUT, DIN), lambda i: (0, 0)),
            pl.BlockSpec((1, DOUT), lambda i: (0, 0)),
        ],
        out_specs=pl.BlockSpec((blk, DH), lambda i: (i, 0)),
        out_shape=jax.ShapeDtypeStruct((N, DH), jnp.float32),
    )(feats, W, b2)


def _pool_body(h_hbm, edge_hbm, out_hbm, idx_all, rows0, rows1, out_v,
               sem0, sem1):
    cid = lax.axis_index("c")
    sid = lax.axis_index("s")
    is0 = cid == 0
    nchunks = jnp.where(is0, CHUNKS0, CHUNKS1)
    chunk_base = jnp.where(is0, sid * CHUNKS0,
                           CORE0_CHUNKS + sid * CHUNKS1)
    node_base = jnp.where(is0, sid * NODES0,
                          CORE0_NODES + sid * NODES1)

    pltpu.sync_copy(edge_hbm.at[pl.ds(chunk_base, CHUNKS0)], idx_all)

    rows = (rows0, rows1)
    sems = (sem0, sem1)
    for b in range(NBUF):
        pltpu.async_copy(h_hbm.at[idx_all.at[b]], rows[b], sems[b])

    inv = jnp.full((LANES,), 1.0 / K, dtype=jnp.float32)

    def step(g, carry):
        for b in range(NBUF):
            c = g * NBUF + b
            r = rows[b]
            pltpu.make_async_copy(h_hbm.at[idx_all.at[c]], r, sems[b]).wait()
            for n in range(CH):
                row = c * CH + n
                for w in range(DH // LANES):
                    sl = pl.ds(w * LANES, LANES)
                    acc_lo = jnp.zeros((LANES,), jnp.float32)
                    acc_hi = jnp.zeros((LANES,), jnp.float32)
                    for j in range(K):
                        packed = plsc.bitcast(r[n * K + j, sl], jnp.bfloat16)
                        lo, hi = plsc.unpack(
                            packed, format=plsc.PackFormat.INTERLEAVED,
                            preferred_element_type=jnp.float32)
                        acc_lo = acc_lo + lo
                        acc_hi = acc_hi + hi
                    out_v[row, sl] = acc_lo * inv
                    out_v[row, pl.ds(DH + w * LANES, LANES)] = acc_hi * inv

            @pl.when(c + NBUF < nchunks)
            def _():
                pltpu.async_copy(h_hbm.at[idx_all.at[c + NBUF]], r, sems[b])
        return carry

    lax.fori_loop(0, nchunks // NBUF, step, None)

    pltpu.sync_copy(out_v.at[pl.ds(0, NODES1)],
                    out_hbm.at[pl.ds(node_base, NODES1)])

    @pl.when(is0)
    def _():
        pltpu.sync_copy(out_v.at[pl.ds(NODES1, NODES0 - NODES1)],
                        out_hbm.at[pl.ds(node_base + NODES1,
                                         NODES0 - NODES1)])


def _pool(h, edge2):
    mesh = plsc.VectorSubcoreMesh(core_axis_name="c", subcore_axis_name="s")
    f = pl.kernel(
        _pool_body,
        out_type=jax.ShapeDtypeStruct((NPAD, DOUT), jnp.float32),
        mesh=mesh,
        compiler_params=pltpu.CompilerParams(needs_layout_passes=False,
                                             use_tc_tiling_on_sc=False),
        scratch_types=[
            pltpu.VMEM((CHUNKS0, CH * K), jnp.int32),
            pltpu.VMEM((CH * K, DH), jnp.float32),
            pltpu.VMEM((CH * K, DH), jnp.float32),
            pltpu.VMEM((NODES0, DOUT), jnp.float32),
            pltpu.SemaphoreType.DMA,
            pltpu.SemaphoreType.DMA,
        ],
    )
    return f(h, edge2)


def kernel(ids, feats, edge_dict, G, ite, W, b):
    h = _fc(feats, W, b.reshape(1, DOUT))
    edge_flat = jnp.pad(edge_dict.reshape(-1), (0, (NPAD - N) * K))
    edge2 = jnp.pad(edge_flat.reshape(NCHUNKS_TOTAL, CH * K),
                    ((0, EDGE_ROWS - NCHUNKS_TOTAL), (0, 0)))
    pooled = _pool(h, edge2)
    return pooled[:N]


# speedup vs baseline: 4.1188x; 1.6654x over previous
import functools

import jax
import jax.numpy as jnp
from jax import lax
from jax.experimental import pallas as pl
from jax.experimental.pallas import tpu as pltpu
from jax.experimental.pallas import tpu_sc as plsc

N = 10000
K = 32
DIN = 128
DOUT = 128
DH = DOUT // 2

NC = 2
NS = 16
NW = NC * NS
NPAD = 10240
CH = 4
NCHUNKS_TOTAL = NPAD // CH
LANES = 16
NBUF = 2

CHUNKS0 = 80
CHUNKS1 = 80
NODES0 = CHUNKS0 * CH
NODES1 = CHUNKS1 * CH
CORE0_NODES = NS * NODES0
CORE0_CHUNKS = NS * CHUNKS0
EDGE_ROWS = 2624


def _fc_body(x_ref, w_ref, b_ref, h_ref):
    acc = lax.dot_general(x_ref[...], w_ref[...],
                          (((1,), (1,)), ((), ())),
                          preferred_element_type=jnp.float32)
    h = jnp.maximum(acc + b_ref[...], 0.0)
    lo = lax.bitcast_convert_type(
        h[:, :DH].astype(jnp.bfloat16), jnp.uint16).astype(jnp.uint32)
    hi = lax.bitcast_convert_type(
        h[:, DH:].astype(jnp.bfloat16), jnp.uint16).astype(jnp.uint32)
    h_ref[...] = lax.bitcast_convert_type((hi << 16) | lo, jnp.float32)


def _fc(feats, W, b2):
    blk = 1000
    return pl.pallas_call(
        _fc_body,
        grid=(N // blk,),
        in_specs=[
            pl.BlockSpec((blk, DIN), lambda i: (i, 0)),
            pl.BlockSpec((DOUT, DIN), lambda i: (0, 0)),
            pl.BlockSpec((1, DOUT), lambda i: (0, 0)),
        ],
        out_specs=pl.BlockSpec((blk, DH), lambda i: (i, 0)),
        out_shape=jax.ShapeDtypeStruct((N, DH), jnp.float32),
    )(feats, W, b2)


STRIPE = 640


def _pool_body(h_hbm, edge_hbm, out_hbm, idx_all, rows0, rows1, out_v,
               h_sh, sem0, sem1):
    cid = lax.axis_index("c")
    sid = lax.axis_index("s")
    is0 = cid == 0
    nchunks = jnp.where(is0, CHUNKS0, CHUNKS1)
    chunk_base = jnp.where(is0, sid * CHUNKS0,
                           CORE0_CHUNKS + sid * CHUNKS1)
    node_base = jnp.where(is0, sid * NODES0,
                          CORE0_NODES + sid * NODES1)

    @pl.when(sid < NS - 1)
    def _():
        pltpu.sync_copy(h_hbm.at[pl.ds(sid * STRIPE, STRIPE)],
                        h_sh.at[pl.ds(sid * STRIPE, STRIPE)])

    @pl.when(sid == NS - 1)
    def _():
        last = N - (NS - 1) * STRIPE
        pltpu.sync_copy(h_hbm.at[pl.ds((NS - 1) * STRIPE, last)],
                        h_sh.at[pl.ds((NS - 1) * STRIPE, last)])

    pltpu.sync_copy(edge_hbm.at[pl.ds(chunk_base, CHUNKS0)], idx_all)

    plsc.subcore_barrier()

    rows = (rows0, rows1)
    sems = (sem0, sem1)
    for b in range(NBUF):
        pltpu.async_copy(h_sh.at[idx_all.at[b]], rows[b], sems[b])

    inv = jnp.full((LANES,), 1.0 / K, dtype=jnp.float32)

    def step(g, carry):
        for b in range(NBUF):
            c = g * NBUF + b
            r = rows[b]
            pltpu.make_async_copy(h_sh.at[idx_all.at[c]], r, sems[b]).wait()
            for n in range(CH):
                row = c * CH + n
                for w in range(DH // LANES):
                    sl = pl.ds(w * LANES, LANES)
                    acc_lo = jnp.zeros((LANES,), jnp.float32)
                    acc_hi = jnp.zeros((LANES,), jnp.float32)
                    for j in range(K):
                        packed = plsc.bitcast(r[n * K + j, sl], jnp.bfloat16)
                        lo, hi = plsc.unpack(
                            packed, format=plsc.PackFormat.INTERLEAVED,
                            preferred_element_type=jnp.float32)
                        acc_lo = acc_lo + lo
                        acc_hi = acc_hi + hi
                    out_v[row, sl] = acc_lo * inv
                    out_v[row, pl.ds(DH + w * LANES, LANES)] = acc_hi * inv

            @pl.when(c + NBUF < nchunks)
            def _():
                pltpu.async_copy(h_sh.at[idx_all.at[c + NBUF]], r, sems[b])
        return carry

    lax.fori_loop(0, nchunks // NBUF, step, None)

    pltpu.sync_copy(out_v.at[pl.ds(0, NODES1)],
                    out_hbm.at[pl.ds(node_base, NODES1)])

    if NODES0 > NODES1:
        @pl.when(is0)
        def _():
            pltpu.sync_copy(out_v.at[pl.ds(NODES1, NODES0 - NODES1)],
                            out_hbm.at[pl.ds(node_base + NODES1,
                                             NODES0 - NODES1)])


def _pool(h, edge2):
    mesh = plsc.VectorSubcoreMesh(core_axis_name="c", subcore_axis_name="s")
    f = pl.kernel(
        _pool_body,
        out_type=jax.ShapeDtypeStruct((NPAD, DOUT), jnp.float32),
        mesh=mesh,
        compiler_params=pltpu.CompilerParams(needs_layout_passes=False,
                                             use_tc_tiling_on_sc=False),
        scratch_types=[
            pltpu.VMEM((CHUNKS0, CH * K), jnp.int32),
            pltpu.VMEM((CH * K, DH), jnp.float32),
            pltpu.VMEM((CH * K, DH), jnp.float32),
            pltpu.VMEM((NODES0, DOUT), jnp.float32),
            pltpu.VMEM_SHARED((N, DH), jnp.float32),
            pltpu.SemaphoreType.DMA,
            pltpu.SemaphoreType.DMA,
        ],
    )
    return f(h, edge2)


def kernel(ids, feats, edge_dict, G, ite, W, b):
    h = _fc(feats, W, b.reshape(1, DOUT))
    edge_flat = jnp.pad(edge_dict.reshape(-1), (0, (NPAD - N) * K))
    edge2 = jnp.pad(edge_flat.reshape(NCHUNKS_TOTAL, CH * K),
                    ((0, EDGE_ROWS - NCHUNKS_TOTAL), (0, 0)))
    pooled = _pool(h, edge2)
    return pooled[:N]
